# baseline (speedup 1.0000x reference)
import functools

import jax
import jax.numpy as jnp
from jax import lax
from jax.experimental import pallas as pl
from jax.experimental.pallas import tpu as pltpu
from jax.experimental.pallas import tpu_sc as plsc

N = 10000
E = 160000
D = 32
K = 1024

NC = 2
NS = 16
NW = NC * NS
NSEG = 2
ESEG = E // NSEG
PER_W = ESEG // NW
CH = 20
NCHUNK = PER_W // CH
MSTAGE = 500
INNER = MSTAGE // CH
FIRE = 5
STRIPE = 624
TAIL = N - NS * STRIPE

_mesh = plsc.VectorSubcoreMesh(core_axis_name="c", subcore_axis_name="s")


def _prep_body(x_ref, w1_ref, b1_ref, g_ref, be_ref, coef_ref):
    x = x_ref[...]
    lane = jax.lax.broadcasted_iota(jnp.int32, x.shape, 1)
    even = (lane % 2) == 0
    xz = jnp.where(even, x, 0.0)
    xr = jnp.where(even, jnp.roll(x, -1, axis=1), 0.0)
    inv_e = 1.0 / E
    st = jnp.sum(x) * inv_e
    ma = jnp.sum(xz) * inv_e
    mb = st - ma
    s2 = jnp.sum(x * x) * inv_e
    saa = jnp.sum(xz * xz) * inv_e
    vaa = saa - ma * ma
    vbb = (s2 - saa) - mb * mb
    vab = jnp.sum(xz * xr) * inv_e - ma * mb
    w0 = w1_ref[0:1, :]
    w1 = w1_ref[1:2, :]
    mean = ma * w0 + mb * w1 + b1_ref[...]
    var = vaa * w0 * w0 + 2.0 * vab * w0 * w1 + vbb * w1 * w1
    s = g_ref[...] * lax.rsqrt(var + 1e-5)
    coef_ref[0:1, :] = s * w0
    coef_ref[1:2, :] = s * w1
    coef_ref[2:3, :] = (b1_ref[...] - mean) * s + be_ref[...]


def _prep(x, w1, b1, g, be):
    return pl.pallas_call(
        _prep_body,
        out_shape=jax.ShapeDtypeStruct((3, K), jnp.float32),
    )(x, w1, b1, g, be)


@functools.partial(
    pl.kernel,
    mesh=_mesh,
    out_type=jax.ShapeDtypeStruct((ESEG, D), jnp.float32),
    scratch_types=[
        pltpu.VMEM((NCHUNK, CH), jnp.int32),
        pltpu.VMEM((MSTAGE, D), jnp.float32),
        pltpu.SemaphoreType.DMA,
    ],
    compiler_params=pltpu.CompilerParams(use_tc_tiling_on_sc=False),
)
def _gather(v_hbm, src_hbm, xj_hbm, idx_v, stage_v, sem):
    wid = lax.axis_index("s") * NC + lax.axis_index("c")
    pltpu.sync_copy(src_hbm.at[wid], idx_v)

    def macro(m, _):
        def fire(f, _):
            handles = []
            for b in range(FIRE):
                j = f * FIRE + b
                handles.append(pltpu.async_copy(
                    v_hbm.at[idx_v.at[m * INNER + j]],
                    stage_v.at[pl.ds(j * CH, CH)], sem))
            for h in handles:
                h.wait()
            return 0

        lax.fori_loop(0, INNER // FIRE, fire, 0)
        pltpu.sync_copy(stage_v,
                        xj_hbm.at[pl.ds(wid * PER_W + m * MSTAGE, MSTAGE)])
        return 0

    lax.fori_loop(0, PER_W // MSTAGE, macro, 0)


def _edge_body(ea_ref, xj_ref, coef_ref, r_ref, s_ref, msg_ref):
    a = ea_ref[:, 0:1]
    b = ea_ref[:, 1:2]
    g = a * coef_ref[0:1, :] + b * coef_ref[1:2, :] + coef_ref[2:3, :]
    h = jnp.tanh(g)
    xr = jnp.dot(xj_ref[...], r_ref[...], preferred_element_type=jnp.float32)
    msg_ref[...] = jnp.dot(xr * h, s_ref[...],
                           preferred_element_type=jnp.float32)


def _edge(ea, xj, coef, r, s, block_e=1600):
    e_seg = ea.shape[0]
    grid = (e_seg // block_e,)
    return pl.pallas_call(
        _edge_body,
        grid=grid,
        in_specs=[
            pl.BlockSpec((block_e, 2), lambda i: (i, 0)),
            pl.BlockSpec((block_e, D), lambda i: (i, 0)),
            pl.BlockSpec((3, K), lambda i: (0, 0)),
            pl.BlockSpec((D, K), lambda i: (0, 0)),
            pl.BlockSpec((K, D), lambda i: (0, 0)),
        ],
        out_specs=pl.BlockSpec((block_e, D), lambda i: (i, 0)),
        out_shape=jax.ShapeDtypeStruct((e_seg, D), jnp.float32),
    )(ea, xj, coef, r, s)


@functools.partial(
    pl.kernel,
    mesh=_mesh,
    out_type=(
        jax.ShapeDtypeStruct((NC, N, D), jnp.float32),
        jax.ShapeDtypeStruct((NC, N, 16), jnp.float32),
    ),
    scratch_types=[
        pltpu.VMEM((NCHUNK, CH), jnp.int32),
        pltpu.VMEM((MSTAGE, D), jnp.float32),
        pltpu.VMEM((CH, 16), jnp.float32),
        pltpu.VMEM_SHARED((N, D), jnp.float32),
        pltpu.VMEM_SHARED((N, 16), jnp.float32),
        pltpu.SemaphoreType.DMA,
    ],
    compiler_params=pltpu.CompilerParams(use_tc_tiling_on_sc=False),
)
def _scatter(msg_hbm, dst_hbm, init32_hbm, init16_hbm, ones_hbm,
             sums_hbm, cnt_hbm, idx_v, stage_v, ones_v, ssum, scnt, sem):
    c = lax.axis_index("c")
    s = lax.axis_index("s")
    wid = s * NC + c

    pltpu.sync_copy(init32_hbm.at[c].at[pl.ds(s * STRIPE, STRIPE)],
                    ssum.at[pl.ds(s * STRIPE, STRIPE)])
    pltpu.sync_copy(init16_hbm.at[c].at[pl.ds(s * STRIPE, STRIPE)],
                    scnt.at[pl.ds(s * STRIPE, STRIPE)])

    @pl.when(s == NS - 1)
    def _seed_tail():
        pltpu.sync_copy(init32_hbm.at[c].at[pl.ds(NS * STRIPE, TAIL)],
                        ssum.at[pl.ds(NS * STRIPE, TAIL)])
        pltpu.sync_copy(init16_hbm.at[c].at[pl.ds(NS * STRIPE, TAIL)],
                        scnt.at[pl.ds(NS * STRIPE, TAIL)])

    pltpu.sync_copy(ones_hbm, ones_v)
    pltpu.sync_copy(dst_hbm.at[wid], idx_v)
    plsc.subcore_barrier()

    def outer(m, _):
        pltpu.sync_copy(msg_hbm.at[pl.ds(wid * PER_W + m * MSTAGE, MSTAGE)],
                        stage_v)

        def batch(f, _):
            handles = []
            for b in range(FIRE):
                j = f * FIRE + b
                jj = m * INNER + j
                handles.append(pltpu.async_copy(
                    stage_v.at[pl.ds(j * CH, CH)],
                    ssum.at[idx_v.at[jj]], sem, add=True))
                handles.append(pltpu.async_copy(
                    ones_v, scnt.at[idx_v.at[jj]], sem, add=True))
            for h in handles:
                h.wait()
            return 0

        lax.fori_loop(0, INNER // FIRE, batch, 0)
        return 0

    lax.fori_loop(0, PER_W // MSTAGE, outer, 0)
    plsc.subcore_barrier()

    pltpu.sync_copy(ssum.at[pl.ds(s * STRIPE, STRIPE)],
                    sums_hbm.at[c].at[pl.ds(s * STRIPE, STRIPE)])
    pltpu.sync_copy(scnt.at[pl.ds(s * STRIPE, STRIPE)],
                    cnt_hbm.at[c].at[pl.ds(s * STRIPE, STRIPE)])

    @pl.when(s == NS - 1)
    def _drain_tail():
        pltpu.sync_copy(ssum.at[pl.ds(NS * STRIPE, TAIL)],
                        sums_hbm.at[c].at[pl.ds(NS * STRIPE, TAIL)])
        pltpu.sync_copy(scnt.at[pl.ds(NS * STRIPE, TAIL)],
                        cnt_hbm.at[c].at[pl.ds(NS * STRIPE, TAIL)])


def _final_body(v_ref, s0_ref, s1_ref, c0_ref, c1_ref, rw_ref, bias_ref, o_ref):
    cnt = jnp.maximum(c0_ref[:, 0:1] + c1_ref[:, 0:1], 1.0)
    aggr = (s0_ref[...] + s1_ref[...]) / cnt
    root = jnp.dot(v_ref[...], rw_ref[...], preferred_element_type=jnp.float32)
    x = aggr + root + bias_ref[...]
    o_ref[...] = jnp.where(x >= 0.0, x, 0.01 * x)


def _final(v, s0, s1, c0, c1, rw, bias, block_n=2000):
    grid = (N // block_n,)
    return pl.pallas_call(
        _final_body,
        grid=grid,
        in_specs=[
            pl.BlockSpec((block_n, D), lambda i: (i, 0)),
            pl.BlockSpec((block_n, D), lambda i: (i, 0)),
            pl.BlockSpec((block_n, D), lambda i: (i, 0)),
            pl.BlockSpec((block_n, 16), lambda i: (i, 0)),
            pl.BlockSpec((block_n, 16), lambda i: (i, 0)),
            pl.BlockSpec((D, D), lambda i: (0, 0)),
            pl.BlockSpec((1, D), lambda i: (0, 0)),
        ],
        out_specs=pl.BlockSpec((block_n, D), lambda i: (i, 0)),
        out_shape=jax.ShapeDtypeStruct((N, D), jnp.float32),
    )(v, s0, s1, c0, c1, rw, bias)


def kernel(v, edge_attr, W1, b1, bn_gamma, bn_beta, root_w, bias, edge_index):
    src = edge_index[0].reshape(NSEG, NW, NCHUNK, CH)
    dst = edge_index[1].reshape(NSEG, NW, NCHUNK, CH)
    eav = edge_attr.reshape(E // 64, 128)

    coef = _prep(eav, W1, b1.reshape(1, K),
                 bn_gamma.reshape(1, K), bn_beta.reshape(1, K))

    k = jnp.arange(K, dtype=jnp.int32)
    r = (k[None, :] // D == jnp.arange(D, dtype=jnp.int32)[:, None])
    r = r.astype(jnp.float32)
    s = (k[:, None] % D == jnp.arange(D, dtype=jnp.int32)[None, :])
    s = s.astype(jnp.float32)

    z32 = jnp.zeros((NC, N, D), jnp.float32)
    z16 = jnp.zeros((NC, N, 16), jnp.float32)
    ones = jnp.ones((CH, 16), jnp.float32)

    ea0 = edge_attr[:ESEG]
    ea1 = edge_attr[ESEG:]

    xj0 = _gather(v, src[0])
    msg0 = _edge(ea0, xj0, coef, r, s)
    xj1 = _gather(v, src[1])
    sums0, cnt0 = _scatter(msg0, dst[0], z32, z16, ones)
    msg1 = _edge(ea1, xj1, coef, r, s)
    sums1, cnt1 = _scatter(msg1, dst[1], sums0, cnt0, ones)

    return _final(v, sums1[0], sums1[1], cnt1[0], cnt1[1],
                  root_w, bias.reshape(1, D))


# speedup vs baseline: 1.1228x; 1.1228x over previous
import functools

import jax
import jax.numpy as jnp
from jax import lax
from jax.experimental import pallas as pl
from jax.experimental.pallas import tpu as pltpu
from jax.experimental.pallas import tpu_sc as plsc

N = 10000
E = 160000
D = 32
K = 1024

NC = 2
NS = 16
NW = NC * NS
ESEG = E
PER_W = ESEG // NW
CH = 40
NCHUNK = PER_W // CH
MSTAGE = 1000
INNER = MSTAGE // CH
FIRE = 5
STRIPE = 624
TAIL = N - NS * STRIPE

_mesh = plsc.VectorSubcoreMesh(core_axis_name="c", subcore_axis_name="s")


def _prep_body(x_ref, w1_ref, b1_ref, g_ref, be_ref, coef_ref):
    x = x_ref[...]
    lane = jax.lax.broadcasted_iota(jnp.int32, x.shape, 1)
    even = (lane % 2) == 0
    xz = jnp.where(even, x, 0.0)
    xr = jnp.where(even, jnp.roll(x, -1, axis=1), 0.0)
    inv_e = 1.0 / E
    st = jnp.sum(x) * inv_e
    ma = jnp.sum(xz) * inv_e
    mb = st - ma
    s2 = jnp.sum(x * x) * inv_e
    saa = jnp.sum(xz * xz) * inv_e
    vaa = saa - ma * ma
    vbb = (s2 - saa) - mb * mb
    vab = jnp.sum(xz * xr) * inv_e - ma * mb
    w0 = w1_ref[0:1, :]
    w1 = w1_ref[1:2, :]
    mean = ma * w0 + mb * w1 + b1_ref[...]
    var = vaa * w0 * w0 + 2.0 * vab * w0 * w1 + vbb * w1 * w1
    s = g_ref[...] * lax.rsqrt(var + 1e-5)
    coef_ref[0:1, :] = s * w0
    coef_ref[1:2, :] = s * w1
    coef_ref[2:3, :] = (b1_ref[...] - mean) * s + be_ref[...]


def _prep(x, w1, b1, g, be):
    return pl.pallas_call(
        _prep_body,
        out_shape=jax.ShapeDtypeStruct((3, K), jnp.float32),
    )(x, w1, b1, g, be)


@functools.partial(
    pl.kernel,
    mesh=_mesh,
    out_type=jax.ShapeDtypeStruct((ESEG, D), jnp.float32),
    scratch_types=[
        pltpu.VMEM((NCHUNK, CH), jnp.int32),
        pltpu.VMEM((MSTAGE, D), jnp.float32),
        pltpu.SemaphoreType.DMA,
    ],
    compiler_params=pltpu.CompilerParams(use_tc_tiling_on_sc=False),
)
def _gather(v_hbm, src_hbm, xj_hbm, idx_v, stage_v, sem):
    wid = lax.axis_index("s") * NC + lax.axis_index("c")
    pltpu.sync_copy(src_hbm.at[wid], idx_v)

    def macro(m, _):
        def fire(f, _):
            handles = []
            for b in range(FIRE):
                j = f * FIRE + b
                handles.append(pltpu.async_copy(
                    v_hbm.at[idx_v.at[m * INNER + j]],
                    stage_v.at[pl.ds(j * CH, CH)], sem))
            for h in handles:
                h.wait()
            return 0

        lax.fori_loop(0, INNER // FIRE, fire, 0)
        pltpu.sync_copy(stage_v,
                        xj_hbm.at[pl.ds(wid * PER_W + m * MSTAGE, MSTAGE)])
        return 0

    lax.fori_loop(0, PER_W // MSTAGE, macro, 0)


def _edge_body(ea_ref, xj_ref, coef_ref, r_ref, s_ref, msg_ref):
    a = ea_ref[:, 0:1]
    b = ea_ref[:, 1:2]
    g = a * coef_ref[0:1, :] + b * coef_ref[1:2, :] + coef_ref[2:3, :]
    h = jnp.tanh(g)
    xr = jnp.dot(xj_ref[...], r_ref[...], preferred_element_type=jnp.float32)
    msg_ref[...] = jnp.dot(xr * h, s_ref[...],
                           preferred_element_type=jnp.float32)


def _edge(ea, xj, coef, r, s, block_e=1280):
    e_seg = ea.shape[0]
    grid = (e_seg // block_e,)
    return pl.pallas_call(
        _edge_body,
        grid=grid,
        in_specs=[
            pl.BlockSpec((block_e, 2), lambda i: (i, 0)),
            pl.BlockSpec((block_e, D), lambda i: (i, 0)),
            pl.BlockSpec((3, K), lambda i: (0, 0)),
            pl.BlockSpec((D, K), lambda i: (0, 0)),
            pl.BlockSpec((K, D), lambda i: (0, 0)),
        ],
        out_specs=pl.BlockSpec((block_e, D), lambda i: (i, 0)),
        out_shape=jax.ShapeDtypeStruct((e_seg, D), jnp.float32),
    )(ea, xj, coef, r, s)


@functools.partial(
    pl.kernel,
    mesh=_mesh,
    out_type=(
        jax.ShapeDtypeStruct((NC, N, D), jnp.float32),
        jax.ShapeDtypeStruct((NC, N, 16), jnp.float32),
    ),
    scratch_types=[
        pltpu.VMEM((NCHUNK, CH), jnp.int32),
        pltpu.VMEM((MSTAGE, D), jnp.float32),
        pltpu.VMEM((CH, 16), jnp.float32),
        pltpu.VMEM_SHARED((N, D), jnp.float32),
        pltpu.VMEM_SHARED((N, 16), jnp.float32),
        pltpu.SemaphoreType.DMA,
    ],
    compiler_params=pltpu.CompilerParams(use_tc_tiling_on_sc=False),
)
def _scatter(msg_hbm, dst_hbm, init32_hbm, init16_hbm, ones_hbm,
             sums_hbm, cnt_hbm, idx_v, stage_v, ones_v, ssum, scnt, sem):
    c = lax.axis_index("c")
    s = lax.axis_index("s")
    wid = s * NC + c

    pltpu.sync_copy(init32_hbm.at[c].at[pl.ds(s * STRIPE, STRIPE)],
                    ssum.at[pl.ds(s * STRIPE, STRIPE)])
    pltpu.sync_copy(init16_hbm.at[c].at[pl.ds(s * STRIPE, STRIPE)],
                    scnt.at[pl.ds(s * STRIPE, STRIPE)])

    @pl.when(s == NS - 1)
    def _seed_tail():
        pltpu.sync_copy(init32_hbm.at[c].at[pl.ds(NS * STRIPE, TAIL)],
                        ssum.at[pl.ds(NS * STRIPE, TAIL)])
        pltpu.sync_copy(init16_hbm.at[c].at[pl.ds(NS * STRIPE, TAIL)],
                        scnt.at[pl.ds(NS * STRIPE, TAIL)])

    pltpu.sync_copy(ones_hbm, ones_v)
    pltpu.sync_copy(dst_hbm.at[wid], idx_v)
    plsc.subcore_barrier()

    def outer(m, _):
        pltpu.sync_copy(msg_hbm.at[pl.ds(wid * PER_W + m * MSTAGE, MSTAGE)],
                        stage_v)

        def batch(f, _):
            handles = []
            for b in range(FIRE):
                j = f * FIRE + b
                jj = m * INNER + j
                handles.append(pltpu.async_copy(
                    stage_v.at[pl.ds(j * CH, CH)],
                    ssum.at[idx_v.at[jj]], sem, add=True))
                handles.append(pltpu.async_copy(
                    ones_v, scnt.at[idx_v.at[jj]], sem, add=True))
            for h in handles:
                h.wait()
            return 0

        lax.fori_loop(0, INNER // FIRE, batch, 0)
        return 0

    lax.fori_loop(0, PER_W // MSTAGE, outer, 0)
    plsc.subcore_barrier()

    pltpu.sync_copy(ssum.at[pl.ds(s * STRIPE, STRIPE)],
                    sums_hbm.at[c].at[pl.ds(s * STRIPE, STRIPE)])
    pltpu.sync_copy(scnt.at[pl.ds(s * STRIPE, STRIPE)],
                    cnt_hbm.at[c].at[pl.ds(s * STRIPE, STRIPE)])

    @pl.when(s == NS - 1)
    def _drain_tail():
        pltpu.sync_copy(ssum.at[pl.ds(NS * STRIPE, TAIL)],
                        sums_hbm.at[c].at[pl.ds(NS * STRIPE, TAIL)])
        pltpu.sync_copy(scnt.at[pl.ds(NS * STRIPE, TAIL)],
                        cnt_hbm.at[c].at[pl.ds(NS * STRIPE, TAIL)])


def _final_body(v_ref, s0_ref, s1_ref, c0_ref, c1_ref, rw_ref, bias_ref, o_ref):
    cnt = jnp.maximum(c0_ref[:, 0:1] + c1_ref[:, 0:1], 1.0)
    aggr = (s0_ref[...] + s1_ref[...]) / cnt
    root = jnp.dot(v_ref[...], rw_ref[...], preferred_element_type=jnp.float32)
    x = aggr + root + bias_ref[...]
    o_ref[...] = jnp.where(x >= 0.0, x, 0.01 * x)


def _final(v, s0, s1, c0, c1, rw, bias, block_n=2000):
    grid = (N // block_n,)
    return pl.pallas_call(
        _final_body,
        grid=grid,
        in_specs=[
            pl.BlockSpec((block_n, D), lambda i: (i, 0)),
            pl.BlockSpec((block_n, D), lambda i: (i, 0)),
            pl.BlockSpec((block_n, D), lambda i: (i, 0)),
            pl.BlockSpec((block_n, 16), lambda i: (i, 0)),
            pl.BlockSpec((block_n, 16), lambda i: (i, 0)),
            pl.BlockSpec((D, D), lambda i: (0, 0)),
            pl.BlockSpec((1, D), lambda i: (0, 0)),
        ],
        out_specs=pl.BlockSpec((block_n, D), lambda i: (i, 0)),
        out_shape=jax.ShapeDtypeStruct((N, D), jnp.float32),
    )(v, s0, s1, c0, c1, rw, bias)


def kernel(v, edge_attr, W1, b1, bn_gamma, bn_beta, root_w, bias, edge_index):
    src = edge_index[0].reshape(NW, NCHUNK, CH)
    dst = edge_index[1].reshape(NW, NCHUNK, CH)
    eav = edge_attr.reshape(E // 64, 128)

    coef = _prep(eav, W1, b1.reshape(1, K),
                 bn_gamma.reshape(1, K), bn_beta.reshape(1, K))

    k = jnp.arange(K, dtype=jnp.int32)
    r = (k[None, :] // D == jnp.arange(D, dtype=jnp.int32)[:, None])
    r = r.astype(jnp.float32)
    s = (k[:, None] % D == jnp.arange(D, dtype=jnp.int32)[None, :])
    s = s.astype(jnp.float32)

    z32 = jnp.zeros((NC, N, D), jnp.float32)
    z16 = jnp.zeros((NC, N, 16), jnp.float32)
    ones = jnp.ones((CH, 16), jnp.float32)

    xj = _gather(v, src)
    msg = _edge(edge_attr, xj, coef, r, s)
    sums_p, cnt_p = _scatter(msg, dst, z32, z16, ones)

    return _final(v, sums_p[0], sums_p[1], cnt_p[0], cnt_p[1],
                  root_w, bias.reshape(1, D))


# speedup vs baseline: 1.2063x; 1.0743x over previous
import functools

import jax
import jax.numpy as jnp
from jax import lax
from jax.experimental import pallas as pl
from jax.experimental.pallas import tpu as pltpu
from jax.experimental.pallas import tpu_sc as plsc

N = 10000
E = 160000
D = 32
K = 1024

NC = 2
NS = 16
NW = NC * NS
ESEG = E
PER_W = ESEG // NW
CH = 40
NCHUNK = PER_W // CH
MSTAGE = 1000
INNER = MSTAGE // CH
FIRE = 5
STRIPE = 624
TAIL = N - NS * STRIPE

_mesh = plsc.VectorSubcoreMesh(core_axis_name="c", subcore_axis_name="s")


def _prep_body(x_ref, w1_ref, b1_ref, g_ref, be_ref, coef_ref):
    x = x_ref[...]
    row = jax.lax.broadcasted_iota(jnp.int32, x.shape, 0)
    even = (row % 2) == 0
    xz = jnp.where(even, x, 0.0)
    xr = jnp.where(even, jnp.roll(x, -1, axis=0), 0.0)
    inv_e = 1.0 / E
    st = jnp.sum(x) * inv_e
    ma = jnp.sum(xz) * inv_e
    mb = st - ma
    s2 = jnp.sum(x * x) * inv_e
    saa = jnp.sum(xz * xz) * inv_e
    vaa = saa - ma * ma
    vbb = (s2 - saa) - mb * mb
    vab = jnp.sum(xz * xr) * inv_e - ma * mb
    w0 = w1_ref[0:1, :]
    w1 = w1_ref[1:2, :]
    mean = ma * w0 + mb * w1 + b1_ref[...]
    var = vaa * w0 * w0 + 2.0 * vab * w0 * w1 + vbb * w1 * w1
    s = g_ref[...] * lax.rsqrt(var + 1e-5)
    coef_ref[0:1, :] = s * w0
    coef_ref[1:2, :] = s * w1
    coef_ref[2:3, :] = (b1_ref[...] - mean) * s + be_ref[...]


def _prep(x, w1, b1, g, be):
    return pl.pallas_call(
        _prep_body,
        out_shape=jax.ShapeDtypeStruct((3, K), jnp.float32),
    )(x, w1, b1, g, be)


@functools.partial(
    pl.kernel,
    mesh=_mesh,
    out_type=jax.ShapeDtypeStruct((ESEG, D), jnp.float32),
    scratch_types=[
        pltpu.VMEM((NCHUNK, CH), jnp.int32),
        pltpu.VMEM((MSTAGE, D), jnp.float32),
        pltpu.SemaphoreType.DMA,
    ],
    compiler_params=pltpu.CompilerParams(use_tc_tiling_on_sc=False),
)
def _gather(v_hbm, src_hbm, xj_hbm, idx_v, stage_v, sem):
    wid = lax.axis_index("s") * NC + lax.axis_index("c")
    pltpu.sync_copy(src_hbm.at[wid], idx_v)

    def macro(m, _):
        def fire(f, _):
            handles = []
            for b in range(FIRE):
                j = f * FIRE + b
                handles.append(pltpu.async_copy(
                    v_hbm.at[idx_v.at[m * INNER + j]],
                    stage_v.at[pl.ds(j * CH, CH)], sem))
            for h in handles:
                h.wait()
            return 0

        lax.fori_loop(0, INNER // FIRE, fire, 0)
        pltpu.sync_copy(stage_v,
                        xj_hbm.at[pl.ds(wid * PER_W + m * MSTAGE, MSTAGE)])
        return 0

    lax.fori_loop(0, PER_W // MSTAGE, macro, 0)


def _edge_body(ea_ref, xj_ref, coef_ref, r_ref, s_ref, msg_ref):
    a = ea_ref[:, 0:1]
    b = ea_ref[:, 1:2]
    g = a * coef_ref[0:1, :] + b * coef_ref[1:2, :] + coef_ref[2:3, :]
    h = jnp.tanh(g)
    xr = jnp.dot(xj_ref[...], r_ref[...], preferred_element_type=jnp.float32)
    msg_ref[...] = jnp.dot(xr * h, s_ref[...],
                           preferred_element_type=jnp.float32)


def _edge(ea, xj, coef, r, s, block_e=1280):
    e_seg = ea.shape[0]
    grid = (e_seg // block_e,)
    return pl.pallas_call(
        _edge_body,
        grid=grid,
        in_specs=[
            pl.BlockSpec((block_e, 2), lambda i: (i, 0)),
            pl.BlockSpec((block_e, D), lambda i: (i, 0)),
            pl.BlockSpec((3, K), lambda i: (0, 0)),
            pl.BlockSpec((D, K), lambda i: (0, 0)),
            pl.BlockSpec((K, D), lambda i: (0, 0)),
        ],
        out_specs=pl.BlockSpec((block_e, D), lambda i: (i, 0)),
        out_shape=jax.ShapeDtypeStruct((e_seg, D), jnp.float32),
    )(ea, xj, coef, r, s)


@functools.partial(
    pl.kernel,
    mesh=_mesh,
    out_type=(
        jax.ShapeDtypeStruct((NC, N, D), jnp.float32),
        jax.ShapeDtypeStruct((NC, N, D), jnp.float32),
    ),
    scratch_types=[
        pltpu.VMEM((NCHUNK, CH), jnp.int32),
        pltpu.VMEM((MSTAGE, D), jnp.float32),
        pltpu.VMEM((CH, D), jnp.float32),
        pltpu.VMEM_SHARED((N, D), jnp.float32),
        pltpu.VMEM_SHARED((N, D), jnp.float32),
        pltpu.SemaphoreType.DMA,
    ],
    compiler_params=pltpu.CompilerParams(use_tc_tiling_on_sc=False),
)
def _scatter(msg_hbm, dst_hbm, init32_hbm, init16_hbm, ones_hbm,
             sums_hbm, cnt_hbm, idx_v, stage_v, ones_v, ssum, scnt, sem):
    c = lax.axis_index("c")
    s = lax.axis_index("s")
    wid = s * NC + c

    pltpu.sync_copy(init32_hbm.at[c].at[pl.ds(s * STRIPE, STRIPE)],
                    ssum.at[pl.ds(s * STRIPE, STRIPE)])
    pltpu.sync_copy(init16_hbm.at[c].at[pl.ds(s * STRIPE, STRIPE)],
                    scnt.at[pl.ds(s * STRIPE, STRIPE)])

    @pl.when(s == NS - 1)
    def _seed_tail():
        pltpu.sync_copy(init32_hbm.at[c].at[pl.ds(NS * STRIPE, TAIL)],
                        ssum.at[pl.ds(NS * STRIPE, TAIL)])
        pltpu.sync_copy(init16_hbm.at[c].at[pl.ds(NS * STRIPE, TAIL)],
                        scnt.at[pl.ds(NS * STRIPE, TAIL)])

    pltpu.sync_copy(ones_hbm, ones_v)
    pltpu.sync_copy(dst_hbm.at[wid], idx_v)
    plsc.subcore_barrier()

    def outer(m, _):
        pltpu.sync_copy(msg_hbm.at[pl.ds(wid * PER_W + m * MSTAGE, MSTAGE)],
                        stage_v)

        def batch(f, _):
            handles = []
            for b in range(FIRE):
                j = f * FIRE + b
                jj = m * INNER + j
                handles.append(pltpu.async_copy(
                    stage_v.at[pl.ds(j * CH, CH)],
                    ssum.at[idx_v.at[jj]], sem, add=True))
                handles.append(pltpu.async_copy(
                    ones_v, scnt.at[idx_v.at[jj]], sem, add=True))
            for h in handles:
                h.wait()
            return 0

        lax.fori_loop(0, INNER // FIRE, batch, 0)
        return 0

    lax.fori_loop(0, PER_W // MSTAGE, outer, 0)
    plsc.subcore_barrier()

    pltpu.sync_copy(ssum.at[pl.ds(s * STRIPE, STRIPE)],
                    sums_hbm.at[c].at[pl.ds(s * STRIPE, STRIPE)])
    pltpu.sync_copy(scnt.at[pl.ds(s * STRIPE, STRIPE)],
                    cnt_hbm.at[c].at[pl.ds(s * STRIPE, STRIPE)])

    @pl.when(s == NS - 1)
    def _drain_tail():
        pltpu.sync_copy(ssum.at[pl.ds(NS * STRIPE, TAIL)],
                        sums_hbm.at[c].at[pl.ds(NS * STRIPE, TAIL)])
        pltpu.sync_copy(scnt.at[pl.ds(NS * STRIPE, TAIL)],
                        cnt_hbm.at[c].at[pl.ds(NS * STRIPE, TAIL)])


def _final_body(v4_ref, s0_ref, s1_ref, c0_ref, c1_ref, rw4_ref, bias4_ref,
                o_ref):
    cnt = jnp.maximum(c0_ref[...] + c1_ref[...], 1.0)
    aggr = (s0_ref[...] + s1_ref[...]) / cnt
    root = jnp.dot(v4_ref[...], rw4_ref[...],
                   preferred_element_type=jnp.float32)
    x = aggr + root + bias4_ref[...]
    o_ref[...] = jnp.where(x >= 0.0, x, 0.01 * x)


def _final(v4, s0, s1, c0, c1, rw4, bias4, block_n=2500):
    n4 = N // 4
    grid = (n4 // block_n,)
    return pl.pallas_call(
        _final_body,
        grid=grid,
        in_specs=[
            pl.BlockSpec((block_n, 4 * D), lambda i: (i, 0)),
            pl.BlockSpec((block_n, 4 * D), lambda i: (i, 0)),
            pl.BlockSpec((block_n, 4 * D), lambda i: (i, 0)),
            pl.BlockSpec((block_n, 4 * D), lambda i: (i, 0)),
            pl.BlockSpec((block_n, 4 * D), lambda i: (i, 0)),
            pl.BlockSpec((4 * D, 4 * D), lambda i: (0, 0)),
            pl.BlockSpec((1, 4 * D), lambda i: (0, 0)),
        ],
        out_specs=pl.BlockSpec((block_n, 4 * D), lambda i: (i, 0)),
        out_shape=jax.ShapeDtypeStruct((n4, 4 * D), jnp.float32),
    )(v4, s0, s1, c0, c1, rw4, bias4)


def kernel(v, edge_attr, W1, b1, bn_gamma, bn_beta, root_w, bias, edge_index):
    src = edge_index[0].reshape(NW, NCHUNK, CH)
    dst = edge_index[1].reshape(NW, NCHUNK, CH)
    eav = jnp.swapaxes(edge_attr.T.reshape(2, E // 128, 128), 0, 1)
    eav = eav.reshape(E // 64, 128)

    coef = _prep(eav, W1, b1.reshape(1, K),
                 bn_gamma.reshape(1, K), bn_beta.reshape(1, K))

    k = jnp.arange(K, dtype=jnp.int32)
    r = (k[None, :] // D == jnp.arange(D, dtype=jnp.int32)[:, None])
    r = r.astype(jnp.float32)
    s = (k[:, None] % D == jnp.arange(D, dtype=jnp.int32)[None, :])
    s = s.astype(jnp.float32)

    z32 = jnp.zeros((NC, N, D), jnp.float32)
    ones = jnp.ones((CH, D), jnp.float32)

    xj = _gather(v, src)
    msg = _edge(edge_attr, xj, coef, r, s)
    sums_p, cnt_p = _scatter(msg, dst, z32, z32, ones)

    n4 = N // 4
    rw4 = jnp.kron(jnp.eye(4, dtype=jnp.float32), root_w)
    bias4 = jnp.tile(bias, 4).reshape(1, 4 * D)
    out4 = _final(v.reshape(n4, 4 * D),
                  sums_p[0].reshape(n4, 4 * D), sums_p[1].reshape(n4, 4 * D),
                  cnt_p[0].reshape(n4, 4 * D), cnt_p[1].reshape(n4, 4 * D),
                  rw4, bias4)
    return out4.reshape(N, D)


# speedup vs baseline: 1.2180x; 1.0097x over previous
import functools

import jax
import jax.numpy as jnp
from jax import lax
from jax.experimental import pallas as pl
from jax.experimental.pallas import tpu as pltpu
from jax.experimental.pallas import tpu_sc as plsc

N = 10000
E = 160000
D = 32
K = 1024

NC = 2
NS = 16
NW = NC * NS
ESEG = E
PER_W = ESEG // NW
CH = 40
NCHUNK = PER_W // CH
MSTAGE = 1000
INNER = MSTAGE // CH
FIRE = 5
STRIPE = 624
TAIL = N - NS * STRIPE

_mesh = plsc.VectorSubcoreMesh(core_axis_name="c", subcore_axis_name="s")


def _prep_body(x_ref, w1_ref, b1_ref, g_ref, be_ref, coef_ref):
    x = x_ref[...]
    row = jax.lax.broadcasted_iota(jnp.int32, x.shape, 0)
    even = (row % 2) == 0
    xz = jnp.where(even, x, 0.0)
    xr = jnp.where(even, jnp.roll(x, -1, axis=0), 0.0)
    inv_e = 1.0 / E
    st = jnp.sum(x) * inv_e
    ma = jnp.sum(xz) * inv_e
    mb = st - ma
    s2 = jnp.sum(x * x) * inv_e
    saa = jnp.sum(xz * xz) * inv_e
    vaa = saa - ma * ma
    vbb = (s2 - saa) - mb * mb
    vab = jnp.sum(xz * xr) * inv_e - ma * mb
    w0 = w1_ref[0:1, :]
    w1 = w1_ref[1:2, :]
    mean = ma * w0 + mb * w1 + b1_ref[...]
    var = vaa * w0 * w0 + 2.0 * vab * w0 * w1 + vbb * w1 * w1
    s = g_ref[...] * lax.rsqrt(var + 1e-5)
    coef_ref[0:1, :] = s * w0
    coef_ref[1:2, :] = s * w1
    coef_ref[2:3, :] = (b1_ref[...] - mean) * s + be_ref[...]


def _prep(x, w1, b1, g, be):
    return pl.pallas_call(
        _prep_body,
        out_shape=jax.ShapeDtypeStruct((3, K), jnp.float32),
    )(x, w1, b1, g, be)


@functools.partial(
    pl.kernel,
    mesh=_mesh,
    out_type=jax.ShapeDtypeStruct((ESEG, D), jnp.float32),
    scratch_types=[
        pltpu.VMEM((NCHUNK, CH), jnp.int32),
        pltpu.VMEM((MSTAGE, D), jnp.float32),
        pltpu.SemaphoreType.DMA,
    ],
    compiler_params=pltpu.CompilerParams(use_tc_tiling_on_sc=False),
)
def _gather(v_hbm, src_hbm, xj_hbm, idx_v, stage_v, sem):
    wid = lax.axis_index("s") * NC + lax.axis_index("c")
    pltpu.sync_copy(src_hbm.at[wid], idx_v)

    def macro(m, _):
        def fire(f, _):
            handles = []
            for b in range(FIRE):
                j = f * FIRE + b
                handles.append(pltpu.async_copy(
                    v_hbm.at[idx_v.at[m * INNER + j]],
                    stage_v.at[pl.ds(j * CH, CH)], sem))
            for h in handles:
                h.wait()
            return 0

        lax.fori_loop(0, INNER // FIRE, fire, 0)
        pltpu.sync_copy(stage_v,
                        xj_hbm.at[pl.ds(wid * PER_W + m * MSTAGE, MSTAGE)])
        return 0

    lax.fori_loop(0, PER_W // MSTAGE, macro, 0)


def _edge_body(eav_ref, w1_ref, b1_ref, g_ref, be_ref,
               ea_ref, xj_ref, r_ref, s_ref, msg_ref, coef_ref):
    @pl.when(pl.program_id(0) == 0)
    def _fold_bn():
        _prep_body(eav_ref, w1_ref, b1_ref, g_ref, be_ref, coef_ref)

    a = ea_ref[:, 0:1]
    b = ea_ref[:, 1:2]
    g = a * coef_ref[0:1, :] + b * coef_ref[1:2, :] + coef_ref[2:3, :]
    h = jnp.tanh(g)
    xr = jnp.dot(xj_ref[...], r_ref[...], preferred_element_type=jnp.float32)
    msg_ref[...] = jnp.dot(xr * h, s_ref[...],
                           preferred_element_type=jnp.float32)


def _edge(eav, w1, b1, g, be, ea, xj, r, s, block_e=1600):
    e_seg = ea.shape[0]
    grid = (e_seg // block_e,)
    return pl.pallas_call(
        _edge_body,
        grid=grid,
        in_specs=[
            pl.BlockSpec((E // 64, 128), lambda i: (0, 0)),
            pl.BlockSpec((2, K), lambda i: (0, 0)),
            pl.BlockSpec((1, K), lambda i: (0, 0)),
            pl.BlockSpec((1, K), lambda i: (0, 0)),
            pl.BlockSpec((1, K), lambda i: (0, 0)),
            pl.BlockSpec((block_e, 2), lambda i: (i, 0)),
            pl.BlockSpec((block_e, D), lambda i: (i, 0)),
            pl.BlockSpec((D, K), lambda i: (0, 0)),
            pl.BlockSpec((K, D), lambda i: (0, 0)),
        ],
        out_specs=pl.BlockSpec((block_e, D), lambda i: (i, 0)),
        out_shape=jax.ShapeDtypeStruct((e_seg, D), jnp.float32),
        scratch_shapes=[pltpu.VMEM((3, K), jnp.float32)],
    )(eav, w1, b1, g, be, ea, xj, r, s)


@functools.partial(
    pl.kernel,
    mesh=_mesh,
    out_type=(
        jax.ShapeDtypeStruct((NC, N, D), jnp.float32),
        jax.ShapeDtypeStruct((NC, N, D), jnp.float32),
    ),
    scratch_types=[
        pltpu.VMEM((NCHUNK, CH), jnp.int32),
        pltpu.VMEM((MSTAGE, D), jnp.float32),
        pltpu.VMEM((CH, D), jnp.float32),
        pltpu.VMEM_SHARED((N, D), jnp.float32),
        pltpu.VMEM_SHARED((N, D), jnp.float32),
        pltpu.SemaphoreType.DMA,
    ],
    compiler_params=pltpu.CompilerParams(use_tc_tiling_on_sc=False),
)
def _scatter(msg_hbm, dst_hbm, init32_hbm, init16_hbm, ones_hbm,
             sums_hbm, cnt_hbm, idx_v, stage_v, ones_v, ssum, scnt, sem):
    c = lax.axis_index("c")
    s = lax.axis_index("s")
    wid = s * NC + c

    pltpu.sync_copy(init32_hbm.at[c].at[pl.ds(s * STRIPE, STRIPE)],
                    ssum.at[pl.ds(s * STRIPE, STRIPE)])
    pltpu.sync_copy(init16_hbm.at[c].at[pl.ds(s * STRIPE, STRIPE)],
                    scnt.at[pl.ds(s * STRIPE, STRIPE)])

    @pl.when(s == NS - 1)
    def _seed_tail():
        pltpu.sync_copy(init32_hbm.at[c].at[pl.ds(NS * STRIPE, TAIL)],
                        ssum.at[pl.ds(NS * STRIPE, TAIL)])
        pltpu.sync_copy(init16_hbm.at[c].at[pl.ds(NS * STRIPE, TAIL)],
                        scnt.at[pl.ds(NS * STRIPE, TAIL)])

    pltpu.sync_copy(ones_hbm, ones_v)
    pltpu.sync_copy(dst_hbm.at[wid], idx_v)
    plsc.subcore_barrier()

    def outer(m, _):
        pltpu.sync_copy(msg_hbm.at[pl.ds(wid * PER_W + m * MSTAGE, MSTAGE)],
                        stage_v)

        def batch(f, _):
            handles = []
            for b in range(FIRE):
                j = f * FIRE + b
                jj = m * INNER + j
                handles.append(pltpu.async_copy(
                    stage_v.at[pl.ds(j * CH, CH)],
                    ssum.at[idx_v.at[jj]], sem, add=True))
                handles.append(pltpu.async_copy(
                    ones_v, scnt.at[idx_v.at[jj]], sem, add=True))
            for h in handles:
                h.wait()
            return 0

        lax.fori_loop(0, INNER // FIRE, batch, 0)
        return 0

    lax.fori_loop(0, PER_W // MSTAGE, outer, 0)
    plsc.subcore_barrier()

    pltpu.sync_copy(ssum.at[pl.ds(s * STRIPE, STRIPE)],
                    sums_hbm.at[c].at[pl.ds(s * STRIPE, STRIPE)])
    pltpu.sync_copy(scnt.at[pl.ds(s * STRIPE, STRIPE)],
                    cnt_hbm.at[c].at[pl.ds(s * STRIPE, STRIPE)])

    @pl.when(s == NS - 1)
    def _drain_tail():
        pltpu.sync_copy(ssum.at[pl.ds(NS * STRIPE, TAIL)],
                        sums_hbm.at[c].at[pl.ds(NS * STRIPE, TAIL)])
        pltpu.sync_copy(scnt.at[pl.ds(NS * STRIPE, TAIL)],
                        cnt_hbm.at[c].at[pl.ds(NS * STRIPE, TAIL)])


def _final_body(v4_ref, s0_ref, s1_ref, c0_ref, c1_ref, rw4_ref, bias4_ref,
                o_ref):
    cnt = jnp.maximum(c0_ref[...] + c1_ref[...], 1.0)
    aggr = (s0_ref[...] + s1_ref[...]) / cnt
    root = jnp.dot(v4_ref[...], rw4_ref[...],
                   preferred_element_type=jnp.float32)
    x = aggr + root + bias4_ref[...]
    o_ref[...] = jnp.where(x >= 0.0, x, 0.01 * x)


def _final(v4, s0, s1, c0, c1, rw4, bias4, block_n=2500):
    n4 = N // 4
    grid = (n4 // block_n,)
    return pl.pallas_call(
        _final_body,
        grid=grid,
        in_specs=[
            pl.BlockSpec((block_n, 4 * D), lambda i: (i, 0)),
            pl.BlockSpec((block_n, 4 * D), lambda i: (i, 0)),
            pl.BlockSpec((block_n, 4 * D), lambda i: (i, 0)),
            pl.BlockSpec((block_n, 4 * D), lambda i: (i, 0)),
            pl.BlockSpec((block_n, 4 * D), lambda i: (i, 0)),
            pl.BlockSpec((4 * D, 4 * D), lambda i: (0, 0)),
            pl.BlockSpec((1, 4 * D), lambda i: (0, 0)),
        ],
        out_specs=pl.BlockSpec((block_n, 4 * D), lambda i: (i, 0)),
        out_shape=jax.ShapeDtypeStruct((n4, 4 * D), jnp.float32),
    )(v4, s0, s1, c0, c1, rw4, bias4)


def kernel(v, edge_attr, W1, b1, bn_gamma, bn_beta, root_w, bias, edge_index):
    src = edge_index[0].reshape(NW, NCHUNK, CH)
    dst = edge_index[1].reshape(NW, NCHUNK, CH)
    eav = jnp.swapaxes(edge_attr.T.reshape(2, E // 128, 128), 0, 1)
    eav = eav.reshape(E // 64, 128)

    k = jnp.arange(K, dtype=jnp.int32)
    r = (k[None, :] // D == jnp.arange(D, dtype=jnp.int32)[:, None])
    r = r.astype(jnp.float32)
    s = (k[:, None] % D == jnp.arange(D, dtype=jnp.int32)[None, :])
    s = s.astype(jnp.float32)

    z32 = jnp.zeros((NC, N, D), jnp.float32)
    ones = jnp.ones((CH, D), jnp.float32)

    xj = _gather(v, src)
    msg = _edge(eav, W1, b1.reshape(1, K), bn_gamma.reshape(1, K),
                bn_beta.reshape(1, K), edge_attr, xj, r, s)
    sums_p, cnt_p = _scatter(msg, dst, z32, z32, ones)

    n4 = N // 4
    rw4 = jnp.kron(jnp.eye(4, dtype=jnp.float32), root_w)
    bias4 = jnp.tile(bias, 4).reshape(1, 4 * D)
    out4 = _final(v.reshape(n4, 4 * D),
                  sums_p[0].reshape(n4, 4 * D), sums_p[1].reshape(n4, 4 * D),
                  cnt_p[0].reshape(n4, 4 * D), cnt_p[1].reshape(n4, 4 * D),
                  rw4, bias4)
    return out4.reshape(N, D)
